# Initial kernel scaffold; baseline (speedup 1.0000x reference)
import functools

import jax
import jax.numpy as jnp
from jax.experimental import pallas as pl

_BLK = 256
_KAPPA = 32.0


def _intrvfl_kernel(x_ref, k_ref, wt_ref, out_ref):
    blk, f_dim = x_ref.shape
    d_dim = k_ref.shape[1]
    thr = jnp.round(x_ref[...] * d_dim)
    iota = jax.lax.broadcasted_iota(jnp.float32, (blk, d_dim), 1)
    acc = jnp.zeros((blk, d_dim), jnp.float32)
    for f in range(f_dim):
        kf = k_ref[f, :][None, :]
        mask = iota < thr[:, f][:, None]
        acc = acc + jnp.where(mask, kf, -kf)
    enc = jnp.clip(acc, -_KAPPA, _KAPPA)
    out_ref[...] = jnp.dot(enc, wt_ref[...], preferred_element_type=jnp.float32)


@jax.jit
def kernel(x, key_hv, therm, W):
    b, f_dim = x.shape
    d_dim = key_hv.shape[1]
    c_dim = W.shape[0]
    wt = W.T
    return pl.pallas_call(
        _intrvfl_kernel,
        grid=(b // _BLK,),
        in_specs=[
            pl.BlockSpec((_BLK, f_dim), lambda i: (i, 0)),
            pl.BlockSpec((f_dim, d_dim), lambda i: (0, 0)),
            pl.BlockSpec((d_dim, c_dim), lambda i: (0, 0)),
        ],
        out_specs=pl.BlockSpec((_BLK, c_dim), lambda i: (i, 0)),
        out_shape=jax.ShapeDtypeStruct((b, c_dim), jnp.float32),
    )(x, key_hv, wt)


# speedup vs baseline: 14.9483x; 14.9483x over previous
import functools

import jax
import jax.numpy as jnp
from jax.experimental import pallas as pl

_BLK = 256
_KAPPA = 32.0


def _intrvfl_kernel(x_ref, k_ref, wt_ref, out_ref):
    blk, f_dim = x_ref.shape
    d_dim = k_ref.shape[1]
    thr = jnp.round(x_ref[...] * d_dim).astype(jnp.int32)
    iota = jax.lax.broadcasted_iota(jnp.int32, (blk, d_dim), 1)
    acc = jnp.zeros((blk, d_dim), jnp.float32)
    for f in range(f_dim):
        kf = k_ref[f, :][None, :]
        mask = iota < thr[:, f][:, None]
        acc = acc + jnp.where(mask, kf, -kf)
    enc = jnp.clip(acc, -_KAPPA, _KAPPA)
    out_ref[...] = jnp.dot(enc, wt_ref[...], preferred_element_type=jnp.float32)


@jax.jit
def kernel(x, key_hv, therm, W):
    b, f_dim = x.shape
    d_dim = key_hv.shape[1]
    c_dim = W.shape[0]
    wt = W.T
    return pl.pallas_call(
        _intrvfl_kernel,
        grid=(b // _BLK,),
        in_specs=[
            pl.BlockSpec((_BLK, f_dim), lambda i: (i, 0)),
            pl.BlockSpec((f_dim, d_dim), lambda i: (0, 0)),
            pl.BlockSpec((d_dim, c_dim), lambda i: (0, 0)),
        ],
        out_specs=pl.BlockSpec((_BLK, c_dim), lambda i: (i, 0)),
        out_shape=jax.ShapeDtypeStruct((b, c_dim), jnp.float32),
    )(x, key_hv, wt)


# speedup vs baseline: 22.8444x; 1.5282x over previous
import functools

import jax
import jax.numpy as jnp
from jax.experimental import pallas as pl

_BLK = 256
_KAPPA = 32.0


def _intrvfl_kernel(x_ref, k_ref, wt_ref, out_ref):
    blk, f_dim = x_ref.shape
    d_dim = k_ref.shape[1]
    thr = jnp.round(x_ref[...] * d_dim).astype(jnp.int16)
    iota = jax.lax.broadcasted_iota(jnp.int16, (blk, d_dim), 1)
    kb = k_ref[...].astype(jnp.bfloat16)
    acc = jnp.zeros((blk, d_dim), jnp.bfloat16)
    for f in range(f_dim):
        kf = kb[f, :][None, :]
        mask = iota < thr[:, f][:, None]
        acc = acc + jnp.where(mask, kf, -kf)
    enc = jnp.clip(acc.astype(jnp.float32), -_KAPPA, _KAPPA)
    out_ref[...] = jnp.dot(enc, wt_ref[...], preferred_element_type=jnp.float32)


@jax.jit
def kernel(x, key_hv, therm, W):
    b, f_dim = x.shape
    d_dim = key_hv.shape[1]
    c_dim = W.shape[0]
    wt = W.T
    return pl.pallas_call(
        _intrvfl_kernel,
        grid=(b // _BLK,),
        in_specs=[
            pl.BlockSpec((_BLK, f_dim), lambda i: (i, 0)),
            pl.BlockSpec((f_dim, d_dim), lambda i: (0, 0)),
            pl.BlockSpec((d_dim, c_dim), lambda i: (0, 0)),
        ],
        out_specs=pl.BlockSpec((_BLK, c_dim), lambda i: (i, 0)),
        out_shape=jax.ShapeDtypeStruct((b, c_dim), jnp.float32),
    )(x, key_hv, wt)


# speedup vs baseline: 24.0943x; 1.0547x over previous
import functools

import jax
import jax.numpy as jnp
from jax.experimental import pallas as pl

_BLK = 256
_KAPPA = 32.0


def _intrvfl_kernel(x_ref, k_ref, w_ref, out_ref):
    blk, f_dim = x_ref.shape
    d_dim = k_ref.shape[1]
    thr = jnp.round(x_ref[...] * d_dim).astype(jnp.int16)
    iota = jax.lax.broadcasted_iota(jnp.int16, (blk, d_dim), 1)
    acc = jnp.zeros((blk, d_dim), jnp.bfloat16)
    for f in range(f_dim):
        kf = k_ref[f, :][None, :]
        mask = iota < thr[:, f][:, None]
        acc = acc + jnp.where(mask, kf, -kf)
    enc = jnp.clip(acc.astype(jnp.float32), -_KAPPA, _KAPPA)
    out_ref[...] = jax.lax.dot_general(
        enc, w_ref[...],
        dimension_numbers=(((1,), (1,)), ((), ())),
        preferred_element_type=jnp.float32,
    )


@jax.jit
def kernel(x, key_hv, therm, W):
    b, f_dim = x.shape
    d_dim = key_hv.shape[1]
    c_dim = W.shape[0]
    kb = key_hv.astype(jnp.bfloat16)
    return pl.pallas_call(
        _intrvfl_kernel,
        grid=(b // _BLK,),
        in_specs=[
            pl.BlockSpec((_BLK, f_dim), lambda i: (i, 0)),
            pl.BlockSpec((f_dim, d_dim), lambda i: (0, 0)),
            pl.BlockSpec((c_dim, d_dim), lambda i: (0, 0)),
        ],
        out_specs=pl.BlockSpec((_BLK, c_dim), lambda i: (i, 0)),
        out_shape=jax.ShapeDtypeStruct((b, c_dim), jnp.float32),
    )(x, kb, W)


# speedup vs baseline: 25.6475x; 1.0645x over previous
import functools

import jax
import jax.numpy as jnp
from jax.experimental import pallas as pl

_BLK = 1024
_KAPPA = 32.0


def _intrvfl_kernel(x_ref, k_ref, w_ref, out_ref):
    blk, f_dim = x_ref.shape
    d_dim = k_ref.shape[1]
    thr = jnp.round(x_ref[...] * d_dim).astype(jnp.int16)
    iota = jax.lax.broadcasted_iota(jnp.int16, (blk, d_dim), 1)
    acc = jnp.zeros((blk, d_dim), jnp.bfloat16)
    for f in range(f_dim):
        kf = k_ref[f, :][None, :]
        mask = iota < thr[:, f][:, None]
        acc = acc + jnp.where(mask, kf, -kf)
    enc = jnp.clip(acc.astype(jnp.float32), -_KAPPA, _KAPPA)
    out_ref[...] = jax.lax.dot_general(
        enc, w_ref[...],
        dimension_numbers=(((1,), (1,)), ((), ())),
        preferred_element_type=jnp.float32,
    )


@jax.jit
def kernel(x, key_hv, therm, W):
    b, f_dim = x.shape
    d_dim = key_hv.shape[1]
    c_dim = W.shape[0]
    kb = key_hv.astype(jnp.bfloat16)
    return pl.pallas_call(
        _intrvfl_kernel,
        grid=(b // _BLK,),
        in_specs=[
            pl.BlockSpec((_BLK, f_dim), lambda i: (i, 0)),
            pl.BlockSpec((f_dim, d_dim), lambda i: (0, 0)),
            pl.BlockSpec((c_dim, d_dim), lambda i: (0, 0)),
        ],
        out_specs=pl.BlockSpec((_BLK, c_dim), lambda i: (i, 0)),
        out_shape=jax.ShapeDtypeStruct((b, c_dim), jnp.float32),
    )(x, kb, W)


# speedup vs baseline: 26.7126x; 1.0415x over previous
import functools

import jax
import jax.numpy as jnp
from jax.experimental import pallas as pl

_BLK = 1024
_KAPPA = 32.0


def _intrvfl_kernel(x_ref, k_ref, w_ref, out_ref):
    blk, f_dim = x_ref.shape
    d_dim = k_ref.shape[1]
    thr = jnp.round(x_ref[...] * d_dim).astype(jnp.int16)
    iota = jax.lax.broadcasted_iota(jnp.int16, (blk, d_dim), 1)
    kb = k_ref[...].astype(jnp.bfloat16)
    acc = jnp.zeros((blk, d_dim), jnp.bfloat16)
    for f in range(f_dim):
        kf = kb[f, :][None, :]
        mask = iota < thr[:, f][:, None]
        acc = acc + jnp.where(mask, kf, -kf)
    enc = jnp.clip(acc, jnp.bfloat16(-_KAPPA), jnp.bfloat16(_KAPPA))
    out_ref[...] = jax.lax.dot_general(
        enc, w_ref[...],
        dimension_numbers=(((1,), (1,)), ((), ())),
        preferred_element_type=jnp.float32,
    )


@jax.jit
def kernel(x, key_hv, therm, W):
    b, f_dim = x.shape
    d_dim = key_hv.shape[1]
    c_dim = W.shape[0]
    return pl.pallas_call(
        _intrvfl_kernel,
        grid=(b // _BLK,),
        in_specs=[
            pl.BlockSpec((_BLK, f_dim), lambda i: (i, 0)),
            pl.BlockSpec((f_dim, d_dim), lambda i: (0, 0)),
            pl.BlockSpec((c_dim, d_dim), lambda i: (0, 0)),
        ],
        out_specs=pl.BlockSpec((_BLK, c_dim), lambda i: (i, 0)),
        out_shape=jax.ShapeDtypeStruct((b, c_dim), jnp.float32),
    )(x, key_hv, W)


# speedup vs baseline: 27.4299x; 1.0269x over previous
import functools

import jax
import jax.numpy as jnp
from jax.experimental import pallas as pl

_BLK = 1024
_KAPPA = 32.0


def _intrvfl_kernel(x_ref, k_ref, w_ref, out_ref):
    blk, f_dim = x_ref.shape
    d_dim = k_ref.shape[1]
    thr = jnp.round(x_ref[...] * d_dim).astype(jnp.int16)
    iota = jax.lax.broadcasted_iota(jnp.int16, (blk, d_dim), 1)
    kb = k_ref[...].astype(jnp.bfloat16)
    kb2 = kb + kb
    ksum = jnp.sum(kb, axis=0)[None, :]
    zero = jnp.zeros((), jnp.bfloat16)
    acc = jnp.zeros((blk, d_dim), jnp.bfloat16)
    for f in range(f_dim):
        kf2 = kb2[f, :][None, :]
        mask = iota < thr[:, f][:, None]
        acc = acc + jnp.where(mask, kf2, zero)
    enc = jnp.clip(acc - ksum, jnp.bfloat16(-_KAPPA), jnp.bfloat16(_KAPPA))
    out_ref[...] = jax.lax.dot_general(
        enc, w_ref[...],
        dimension_numbers=(((1,), (1,)), ((), ())),
        preferred_element_type=jnp.float32,
    )


@jax.jit
def kernel(x, key_hv, therm, W):
    b, f_dim = x.shape
    d_dim = key_hv.shape[1]
    c_dim = W.shape[0]
    return pl.pallas_call(
        _intrvfl_kernel,
        grid=(b // _BLK,),
        in_specs=[
            pl.BlockSpec((_BLK, f_dim), lambda i: (i, 0)),
            pl.BlockSpec((f_dim, d_dim), lambda i: (0, 0)),
            pl.BlockSpec((c_dim, d_dim), lambda i: (0, 0)),
        ],
        out_specs=pl.BlockSpec((_BLK, c_dim), lambda i: (i, 0)),
        out_shape=jax.ShapeDtypeStruct((b, c_dim), jnp.float32),
    )(x, key_hv, W)
